# Initial kernel scaffold; baseline (speedup 1.0000x reference)
import jax
import jax.numpy as jnp
from jax.experimental import pallas as pl


def kernel(x, edge_index, W, b, gamma, beta):
    raise NotImplementedError("write your pallas kernel here")


# speedup vs baseline: 25.2760x; 25.2760x over previous
import dataclasses
import functools

import jax
import jax.numpy as jnp
from jax import lax
from jax.experimental import pallas as pl
from jax.experimental.pallas import tpu as pltpu
from jax.experimental.pallas import tpu_sc as plsc

N = 10000
E = 320000
D = 128
EPS = 1e-5

NC = 2
NS = 16
NW = NC * NS
EPW = E // NW

NHR = 640
NHF = NHR * 16

CH = 80
NCHUNK = EPW // CH
NPAD = 10240
RPT = NPAD // NS
NZR = 160

RB = 2000


def _sc_compiler_params():
    cp = pltpu.CompilerParams()
    if "needs_layout_passes" in pltpu.CompilerParams.__dataclass_fields__:
        cp = dataclasses.replace(cp, needs_layout_passes=False)
    return cp


def _sc_hist(dst_rows):
    mesh = plsc.VectorSubcoreMesh(core_axis_name="c", subcore_axis_name="s")

    @functools.partial(
        pl.kernel,
        out_type=jax.ShapeDtypeStruct((NW, NHR, 16), jnp.float32),
        mesh=mesh,
        compiler_params=_sc_compiler_params(),
        scratch_types=[
            pltpu.VMEM((EPW,), jnp.int32),
            pltpu.VMEM((NHR, 16), jnp.float32),
            pltpu.SemaphoreType.DMA,
        ],
    )
    def hist_kernel(dst_hbm, out_hbm, idx_v, hist_v, sem):
        cid = lax.axis_index("c")
        sid = lax.axis_index("s")
        wid = sid * NC + cid
        pltpu.async_copy(dst_hbm.at[wid], idx_v, sem).wait()

        zeros16 = jnp.zeros((16,), jnp.float32)

        @pl.loop(0, NHR)
        def _(i):
            hist_v[i] = zeros16

        ones16 = jnp.ones((16,), jnp.float32)

        @pl.loop(0, EPW // 16)
        def _(i):
            iv = idx_v[pl.ds(i * 16, 16)]
            row = lax.shift_right_logical(iv, 4)
            lane = lax.bitwise_and(iv, 15)
            plsc.addupdate_scatter(hist_v, [row, lane], ones16)

        pltpu.sync_copy(hist_v, out_hbm.at[wid])

    return hist_kernel(dst_rows)


def _tc_dinv(degp2):
    def body(p_ref, o_ref):
        s = jnp.sum(p_ref[...], axis=0, keepdims=True) + 1.0
        o_ref[...] = lax.rsqrt(s)

    return pl.pallas_call(
        body,
        grid=(NHF // 1280,),
        in_specs=[pl.BlockSpec((NW, 1280), lambda i: (0, i))],
        out_specs=pl.BlockSpec((1, 1280), lambda i: (0, i)),
        out_shape=jax.ShapeDtypeStruct((1, NHF), jnp.float32),
    )(degp2)


def _tc_matmul_scale(x, W, dinv_col):
    def body(x_ref, w_ref, d_ref, o_ref):
        h = jnp.dot(x_ref[...], w_ref[...], preferred_element_type=jnp.float32)
        o_ref[...] = h * d_ref[...]

    return pl.pallas_call(
        body,
        grid=(N // RB,),
        in_specs=[
            pl.BlockSpec((RB, D), lambda i: (i, 0)),
            pl.BlockSpec((D, D), lambda i: (0, 0)),
            pl.BlockSpec((RB, 1), lambda i: (i, 0)),
        ],
        out_specs=pl.BlockSpec((RB, D), lambda i: (i, 0)),
        out_shape=jax.ShapeDtypeStruct((N, D), jnp.float32),
    )(x, W, dinv_col)


def _sc_agg(hp, src_r, dst_r, zrows):
    mesh = plsc.VectorSubcoreMesh(core_axis_name="c", subcore_axis_name="s")

    @functools.partial(
        pl.kernel,
        out_type=jax.ShapeDtypeStruct((NC, NPAD, D), jnp.float32),
        mesh=mesh,
        compiler_params=_sc_compiler_params(),
        scratch_types=[
            pltpu.VMEM((NCHUNK, CH), jnp.int32),
            pltpu.VMEM((NCHUNK, CH), jnp.int32),
            pltpu.VMEM((CH, D), jnp.float32),
            pltpu.VMEM_SHARED((NPAD, D), jnp.float32),
            pltpu.SemaphoreType.DMA,
        ],
    )
    def agg_kernel(hp_hbm, src_hbm, dst_hbm, z_hbm, out_hbm,
                   sidx_v, didx_v, rows_v, S_sh, sem):
        cid = lax.axis_index("c")
        sid = lax.axis_index("s")
        wid = sid * NC + cid

        pltpu.async_copy(src_hbm.at[wid], sidx_v, sem).wait()
        pltpu.async_copy(dst_hbm.at[wid], didx_v, sem).wait()

        @pl.loop(0, RPT // NZR)
        def _(i):
            pltpu.sync_copy(z_hbm, S_sh.at[pl.ds(sid * RPT + i * NZR, NZR)])

        plsc.subcore_barrier()

        @pl.loop(0, NCHUNK)
        def _(j):
            pltpu.async_copy(hp_hbm.at[sidx_v.at[j]], rows_v, sem).wait()
            pltpu.sync_copy(rows_v, S_sh.at[didx_v.at[j]], add=True)

        plsc.subcore_barrier()

        @pl.loop(0, RPT // NZR)
        def _(i):
            pltpu.sync_copy(
                S_sh.at[pl.ds(sid * RPT + i * NZR, NZR)],
                out_hbm.at[cid, pl.ds(sid * RPT + i * NZR, NZR)],
            )

    return agg_kernel(hp, src_r, dst_r, zrows)


def _tc_epilogue(Sp, hp, dinv_col, b2, g2, be2):
    def body(S_ref, hp_ref, d_ref, b_ref, g_ref, be_ref, o_ref, acc_ref):
        p = pl.program_id(0)
        i = pl.program_id(1)
        t = (S_ref[0] + S_ref[1] + hp_ref[...]) * d_ref[...] + b_ref[...]
        t = jnp.maximum(t, 0.0)

        @pl.when(jnp.logical_and(p == 0, i == 0))
        def _():
            acc_ref[...] = jnp.zeros_like(acc_ref)

        @pl.when(p == 0)
        def _():
            acc_ref[0:1, :] += jnp.sum(t, axis=0, keepdims=True)
            acc_ref[1:2, :] += jnp.sum(t * t, axis=0, keepdims=True)

        @pl.when(p == 1)
        def _():
            mean = acc_ref[0:1, :] * (1.0 / N)
            var = acc_ref[1:2, :] * (1.0 / N) - mean * mean
            o_ref[...] = ((t - mean) * lax.rsqrt(var + EPS) * g_ref[...]
                          + be_ref[...])

    return pl.pallas_call(
        body,
        grid=(2, N // RB),
        in_specs=[
            pl.BlockSpec((NC, RB, D), lambda p, i: (0, i, 0)),
            pl.BlockSpec((RB, D), lambda p, i: (i, 0)),
            pl.BlockSpec((RB, 1), lambda p, i: (i, 0)),
            pl.BlockSpec((1, D), lambda p, i: (0, 0)),
            pl.BlockSpec((1, D), lambda p, i: (0, 0)),
            pl.BlockSpec((1, D), lambda p, i: (0, 0)),
        ],
        out_specs=pl.BlockSpec((RB, D), lambda p, i: (i, 0)),
        out_shape=jax.ShapeDtypeStruct((N, D), jnp.float32),
        scratch_shapes=[pltpu.VMEM((2, D), jnp.float32)],
    )(Sp, hp, dinv_col, b2, g2, be2)


def kernel(x, edge_index, W, b, gamma, beta):
    src = edge_index[0]
    dst = edge_index[1]

    degp = _sc_hist(dst.reshape(NW, EPW))
    dinv_row = _tc_dinv(degp.reshape(NW, NHF))
    dinv_col = dinv_row.reshape(NHF, 1)[:N]

    hp = _tc_matmul_scale(x, W, dinv_col)

    Sp = _sc_agg(
        hp,
        src.reshape(NW, NCHUNK, CH),
        dst.reshape(NW, NCHUNK, CH),
        jnp.zeros((NZR, D), jnp.float32),
    )

    return _tc_epilogue(
        Sp, hp, dinv_col,
        b.reshape(1, D), gamma.reshape(1, D), beta.reshape(1, D),
    )
